# baseline (speedup 1.0000x reference)
import jax
import jax.numpy as jnp
from jax.experimental import pallas as pl
from jax.experimental.pallas import tpu as pltpu

_T = 16384
_D = 2048
_E = 64
_K = 2
_BT = 1024


def _router_body(x_ref, w_ref, aux_ref, probst_ref):
    logits = jax.lax.dot_general(
        w_ref[...], x_ref[...], (((0,), (1,)), ((), ())),
        preferred_element_type=jnp.float32)

    m1 = jnp.max(logits, axis=0, keepdims=True)
    e = jnp.exp(logits - m1)
    s = jnp.sum(e, axis=0, keepdims=True)
    probst_ref[...] = e * (1.0 / s)

    iota = jax.lax.broadcasted_iota(jnp.int32, logits.shape, 0).astype(jnp.float32)
    i1 = jnp.min(jnp.where(logits == m1, iota, float(_E)), axis=0, keepdims=True)
    masked = jnp.where(iota == i1, -jnp.inf, logits)
    l2 = jnp.max(masked, axis=0, keepdims=True)
    i2 = jnp.min(jnp.where(masked == l2, iota, float(_E)), axis=0, keepdims=True)

    e2 = jnp.exp(l2 - m1)
    t1 = 1.0 / (1.0 + e2 + 1e-9 * s)
    aux_ref[...] = jnp.concatenate(
        [t1, e2 * t1, i1, i2, jnp.zeros((4, t1.shape[1]), jnp.float32)], axis=0)


@jax.jit
def kernel(x, W_gate):
    aux, probst = pl.pallas_call(
        _router_body,
        grid=(_T // _BT,),
        in_specs=[
            pl.BlockSpec((_BT, _D), lambda i: (i, 0)),
            pl.BlockSpec((_D, _E), lambda i: (0, 0)),
        ],
        out_specs=[
            pl.BlockSpec((8, _BT), lambda i: (0, i)),
            pl.BlockSpec((_E, _BT), lambda i: (0, i)),
        ],
        out_shape=[
            jax.ShapeDtypeStruct((8, _T), jnp.float32),
            jax.ShapeDtypeStruct((_E, _T), jnp.float32),
        ],
        compiler_params=pltpu.CompilerParams(
            dimension_semantics=("arbitrary",),
        ),
    )(x, W_gate)
    tkp = aux[0:2].T
    tki = aux[2:4].T.astype(jnp.int32)
    return (tkp, tki, probst.T)
